# Initial kernel scaffold; baseline (speedup 1.0000x reference)
import jax
import jax.numpy as jnp
from jax.experimental import pallas as pl


def kernel(node_feat, edge_index, W1, b1, g1, be1, W2, b2, g2, be2):
    raise NotImplementedError("write your pallas kernel here")


# speedup vs baseline: 3.4544x; 3.4544x over previous
import functools

import jax
import jax.numpy as jnp
import numpy as np
from jax import lax
from jax.experimental import pallas as pl
from jax.experimental.pallas import tpu as pltpu
from jax.experimental.pallas import tpu_sc as plsc

N = 10000
E = 160000
D = 128
OUT = 128
DP = 32
KRON = 400

NC = 2
NS = 16
NW = NC * NS
CH = 128
NCHUNK = E // CH
CHUNKS_PER_CORE = NCHUNK // NC
NP = 10240
ROWS_PER_TILE = NP // NS


def _node_proj_body(x_ref, w_ref, b_ref, g_ref, be_ref, o_ref):
    y = jnp.dot(x_ref[...], w_ref[...], preferred_element_type=jnp.float32)
    y = y + b_ref[...]
    mu = jnp.sum(y, axis=1, keepdims=True) * (1.0 / 20.0)
    var = jnp.sum(y * y, axis=1, keepdims=True) * (1.0 / 20.0) - mu * mu
    h = (y - mu) * lax.rsqrt(var + 1e-5) * g_ref[...] + be_ref[...]
    o_ref[...] = jnp.maximum(h, 0.0)


def _node_proj(x, w1p, b1p, g1p, be1p):
    blk = 2000
    grid = N // blk
    return pl.pallas_call(
        _node_proj_body,
        grid=(grid,),
        in_specs=[
            pl.BlockSpec((blk, D), lambda i: (i, 0)),
            pl.BlockSpec((D, DP), lambda i: (0, 0)),
            pl.BlockSpec((1, DP), lambda i: (0, 0)),
            pl.BlockSpec((1, DP), lambda i: (0, 0)),
            pl.BlockSpec((1, DP), lambda i: (0, 0)),
        ],
        out_specs=pl.BlockSpec((blk, DP), lambda i: (i, 0)),
        out_shape=jax.ShapeDtypeStruct((N, DP), jnp.float32),
    )(x, w1p, b1p, g1p, be1p)


_MESH = plsc.VectorSubcoreMesh(
    core_axis_name="c", subcore_axis_name="s", num_cores=NC, num_subcores=NS)


@functools.partial(
    pl.kernel,
    out_type=(
        jax.ShapeDtypeStruct((E, DP), jnp.float32),
        jax.ShapeDtypeStruct((E, DP), jnp.float32),
    ),
    mesh=_MESH,
    scratch_types=[
        pltpu.VMEM((CH,), jnp.int32),
        pltpu.VMEM((CH,), jnp.int32),
        pltpu.VMEM((CH, DP), jnp.float32),
        pltpu.VMEM((CH, DP), jnp.float32),
        pltpu.SemaphoreType.DMA,
        pltpu.SemaphoreType.DMA,
    ],
    compiler_params=pltpu.CompilerParams(use_tc_tiling_on_sc=False),
)
def _gather_sc(h_hbm, eis_hbm, eid_hbm, srcg_hbm, dstg_hbm,
               idxs_v, idxd_v, rows_s, rows_d, sem_s, sem_d):
    c = lax.axis_index("c")
    s = lax.axis_index("s")
    wid = s * NC + c

    def body(t, carry):
        ch = wid + t * NW

        @pl.when(ch < NCHUNK)
        def _():
            off = pl.multiple_of(ch * CH, CH)
            pltpu.sync_copy(eis_hbm.at[pl.ds(off, CH)], idxs_v)
            pltpu.sync_copy(eid_hbm.at[pl.ds(off, CH)], idxd_v)
            cps = pltpu.async_copy(h_hbm.at[idxs_v], rows_s, sem_s)
            cpd = pltpu.async_copy(h_hbm.at[idxd_v], rows_d, sem_d)
            cps.wait()
            cpd.wait()
            pltpu.sync_copy(rows_s, srcg_hbm.at[pl.ds(off, CH)])
            pltpu.sync_copy(rows_d, dstg_hbm.at[pl.ds(off, CH)])

        return carry

    lax.fori_loop(0, (NCHUNK + NW - 1) // NW, body, 0)


def _edge_body(srcg_ref, dstg_ref, r_ref, s_ref, w2_ref, b2_ref, g2_ref,
               be2_ref, o_ref):
    a = jnp.dot(srcg_ref[...], r_ref[...], preferred_element_type=jnp.float32)
    b = jnp.dot(dstg_ref[...], s_ref[...], preferred_element_type=jnp.float32)
    kron = a * b
    y = jnp.dot(kron, w2_ref[...], preferred_element_type=jnp.float32)
    y = y + b2_ref[...]
    mu = jnp.mean(y, axis=1, keepdims=True)
    var = jnp.mean(y * y, axis=1, keepdims=True) - mu * mu
    h = (y - mu) * lax.rsqrt(var + 1e-5) * g2_ref[...] + be2_ref[...]
    o_ref[...] = jnp.maximum(h, 0.0)


def _edge_mlp(srcg, dstg, rmat, smat, w2, b2, g2, be2):
    blk = 1280
    grid = E // blk
    return pl.pallas_call(
        _edge_body,
        grid=(grid,),
        in_specs=[
            pl.BlockSpec((blk, DP), lambda i: (i, 0)),
            pl.BlockSpec((blk, DP), lambda i: (i, 0)),
            pl.BlockSpec((DP, KRON), lambda i: (0, 0)),
            pl.BlockSpec((DP, KRON), lambda i: (0, 0)),
            pl.BlockSpec((KRON, OUT), lambda i: (0, 0)),
            pl.BlockSpec((1, OUT), lambda i: (0, 0)),
            pl.BlockSpec((1, OUT), lambda i: (0, 0)),
            pl.BlockSpec((1, OUT), lambda i: (0, 0)),
        ],
        out_specs=pl.BlockSpec((blk, OUT), lambda i: (i, 0)),
        out_shape=jax.ShapeDtypeStruct((E, OUT), jnp.float32),
    )(srcg, dstg, rmat, smat, w2, b2, g2, be2)


@functools.partial(
    pl.kernel,
    out_type=(
        jax.ShapeDtypeStruct((NP, OUT), jnp.float32),
        jax.ShapeDtypeStruct((NP, OUT), jnp.float32),
    ),
    mesh=_MESH,
    scratch_types=[
        pltpu.VMEM_SHARED((NP, OUT), jnp.float32),
        pltpu.VMEM((CH, OUT), jnp.float32),
        pltpu.VMEM((CH, OUT), jnp.float32),
        pltpu.VMEM((CH,), jnp.int32),
    ],
)
def _scatter_sc(m_hbm, eid_hbm, zrows_hbm, p0_hbm, p1_hbm, acc, zbuf, mv, idxv):
    c = lax.axis_index("c")
    s = lax.axis_index("s")
    pltpu.sync_copy(zrows_hbm, zbuf)
    for j in range(ROWS_PER_TILE // CH):
        pltpu.sync_copy(zbuf, acc.at[pl.ds(s * ROWS_PER_TILE + j * CH, CH)])
    plsc.subcore_barrier()

    def body(t, carry):
        k = s + t * NS

        @pl.when(k < CHUNKS_PER_CORE)
        def _():
            ch = c * CHUNKS_PER_CORE + k
            off = pl.multiple_of(ch * CH, CH)
            pltpu.sync_copy(eid_hbm.at[pl.ds(off, CH)], idxv)
            pltpu.sync_copy(m_hbm.at[pl.ds(off, CH)], mv)
            pltpu.sync_copy(mv, acc.at[idxv], add=True)

        return carry

    lax.fori_loop(0, (CHUNKS_PER_CORE + NS - 1) // NS, body, 0)
    plsc.subcore_barrier()
    for j in range(ROWS_PER_TILE // CH):
        row = s * ROWS_PER_TILE + j * CH
        pltpu.sync_copy(acc.at[pl.ds(row, CH)], zbuf)

        @pl.when(c == 0)
        def _():
            pltpu.sync_copy(zbuf, p0_hbm.at[pl.ds(row, CH)])

        @pl.when(c == 1)
        def _():
            pltpu.sync_copy(zbuf, p1_hbm.at[pl.ds(row, CH)])


def _combine_body(p0_ref, p1_ref, o_ref):
    o_ref[...] = p0_ref[...] + p1_ref[...]


def _combine(p0, p1):
    blk = 2000
    grid = N // blk
    return pl.pallas_call(
        _combine_body,
        grid=(grid,),
        in_specs=[
            pl.BlockSpec((blk, OUT), lambda i: (i, 0)),
            pl.BlockSpec((blk, OUT), lambda i: (i, 0)),
        ],
        out_specs=pl.BlockSpec((blk, OUT), lambda i: (i, 0)),
        out_shape=jax.ShapeDtypeStruct((N, OUT), jnp.float32),
    )(p0, p1)


def _build_rs():
    r = np.zeros((DP, KRON), np.float32)
    s = np.zeros((DP, KRON), np.float32)
    for a in range(20):
        for k in range(20):
            r[a, a * 20 + k] = 1.0
            s[k, a * 20 + k] = 1.0
    return r, s


_R_NP, _S_NP = _build_rs()


def kernel(node_feat, edge_index, W1, b1, g1, be1, W2, b2, g2, be2):
    w1p = jnp.pad(W1, ((0, 0), (0, DP - 20)))
    b1p = jnp.pad(b1, (0, DP - 20)).reshape(1, DP)
    g1p = jnp.pad(g1, (0, DP - 20)).reshape(1, DP)
    be1p = jnp.pad(be1, (0, DP - 20)).reshape(1, DP)
    ei_src = edge_index[0]
    ei_dst = edge_index[1]
    zrows = jnp.zeros((CH, OUT), jnp.float32)

    h32 = _node_proj(node_feat, w1p, b1p, g1p, be1p)
    srcg, dstg = _gather_sc(h32, ei_src, ei_dst)
    m = _edge_mlp(srcg, dstg, jnp.asarray(_R_NP), jnp.asarray(_S_NP), W2,
                  b2.reshape(1, OUT), g2.reshape(1, OUT), be2.reshape(1, OUT))
    p0, p1 = _scatter_sc(m, ei_dst, zrows)
    return _combine(p0, p1)
